# baseline (speedup 1.0000x reference)
import jax
import jax.numpy as jnp
from jax.experimental import pallas as pl
from jax.experimental.pallas import tpu as pltpu

_B, _R = 4, 100
_HW = 784
_C = 81
_PPB = 112
_G = _HW // _PPB
_CB = _C * _B
_ROWS = _PPB * _CB
_QR = _PPB * _B
_PCH = 2
_CHR = _PCH * _CB
_CHQ = _PCH * _B


def _loss_kernel(cls_ref, pred_a_ref, pred_b_ref, pred_c_ref, pred_d_ref, tm_ref, out_ref, oh_ref, vm_ref, acc_ref):
    g = pl.program_id(0)

    @pl.when(g == 0)
    def _():
        cls = cls_ref[...]
        cid = jax.lax.broadcasted_iota(jnp.int32, (_C, _B, _R), 0)
        oh1 = (cid == cls[None, :, :]).astype(jnp.float32).reshape(_CB, _R)
        vm1 = (cls > 0).astype(jnp.float32)
        for q in range(_PCH):
            oh_ref[pl.ds(q * _CB, _CB), :] = oh1
            vm_ref[pl.ds(q * _B, _B), :] = vm1
        acc_ref[...] = jnp.zeros((_CHQ, _R), jnp.float32)

    eps = jnp.float32(1e-7)
    one = jnp.float32(1.0)
    oh = oh_ref[...]
    vm = vm_ref[...]
    quarter = _PPB // _PCH // 4
    prefs = (pred_a_ref, pred_b_ref, pred_c_ref, pred_d_ref)
    for q in range(_PPB // _PCH):
        pref = prefs[q // quarter]
        qq = q % quarter
        xm = pref[pl.ds(qq * _CHR, _CHR), :] * oh
        yp = jnp.sum(xm.reshape(_PCH, _C, _B, _R), axis=1)
        p = jnp.clip(yp.reshape(_CHQ, _R), eps, one - eps)
        y = tm_ref[pl.ds(q * _CHQ, _CHQ), :]
        bce = -(y * jnp.log(p) + (one - y) * jnp.log(one - p))
        acc_ref[...] += bce * vm

    @pl.when(g == _G - 1)
    def _():
        cnt = jnp.sum((cls_ref[...] > 0).astype(jnp.float32))
        denom = cnt * jnp.float32(_HW)
        total = jnp.sum(acc_ref[...])
        out_ref[0, 0] = jnp.where(cnt > 0, total / denom, jnp.float32(0.0))


def kernel(target_masks, target_class_ids, pred_masks):
    pred_v = jnp.transpose(pred_masks, (2, 3, 4, 0, 1)).reshape(_HW * _CB, _R)
    tm_v = jnp.transpose(target_masks, (2, 3, 0, 1)).reshape(_HW * _B, _R)

    loss = pl.pallas_call(
        _loss_kernel,
        grid=(_G,),
        in_specs=[
            pl.BlockSpec((_B, _R), lambda g: (0, 0)),
            pl.BlockSpec((_ROWS // 4, _R), lambda g: (4 * g, 0)),
            pl.BlockSpec((_ROWS // 4, _R), lambda g: (4 * g + 1, 0)),
            pl.BlockSpec((_ROWS // 4, _R), lambda g: (4 * g + 2, 0)),
            pl.BlockSpec((_ROWS // 4, _R), lambda g: (4 * g + 3, 0)),
            pl.BlockSpec((_QR, _R), lambda g: (g, 0)),
        ],
        out_specs=pl.BlockSpec(memory_space=pltpu.SMEM),
        out_shape=jax.ShapeDtypeStruct((1, 1), jnp.float32),
        scratch_shapes=[
            pltpu.VMEM((_CHR, _R), jnp.float32),
            pltpu.VMEM((_CHQ, _R), jnp.float32),
            pltpu.VMEM((_CHQ, _R), jnp.float32),
        ],
    )(target_class_ids, pred_v, pred_v, pred_v, pred_v, tm_v)
    return loss[0, 0]


# speedup vs baseline: 1.0044x; 1.0044x over previous
import jax
import jax.numpy as jnp
from jax.experimental import pallas as pl
from jax.experimental.pallas import tpu as pltpu

_B, _R = 4, 100
_HW = 784
_C = 81
_PPB = 112
_G = _HW // _PPB
_CB = _C * _B
_ROWS = _PPB * _CB
_QR = _PPB * _B
_PCH = 2
_CHR = _PCH * _CB
_CHQ = _PCH * _B


def _loss_kernel(cls_ref, pred_a_ref, pred_b_ref, tm_ref, out_ref, oh_ref, vm_ref, acc_ref):
    g = pl.program_id(0)

    @pl.when(g == 0)
    def _():
        cls = cls_ref[...]
        cid = jax.lax.broadcasted_iota(jnp.int32, (_C, _B, _R), 0)
        oh1 = (cid == cls[None, :, :]).astype(jnp.float32).reshape(_CB, _R)
        vm1 = (cls > 0).astype(jnp.float32)
        for q in range(_PCH):
            oh_ref[pl.ds(q * _CB, _CB), :] = oh1
            vm_ref[pl.ds(q * _B, _B), :] = vm1
        acc_ref[...] = jnp.zeros((_CHQ, _R), jnp.float32)

    eps = jnp.float32(1e-7)
    one = jnp.float32(1.0)
    oh = oh_ref[...]
    vm = vm_ref[...]
    half = _PPB // _PCH // 2
    for q in range(_PPB // _PCH):
        pref = pred_a_ref if q < half else pred_b_ref
        qq = q if q < half else q - half
        xm = pref[pl.ds(qq * _CHR, _CHR), :] * oh
        yp = jnp.sum(xm.reshape(_PCH, _C, _B, _R), axis=1)
        p = jnp.clip(yp.reshape(_CHQ, _R), eps, one - eps)
        y = tm_ref[pl.ds(q * _CHQ, _CHQ), :]
        bce = -(y * jnp.log(p) + (one - y) * jnp.log(one - p))
        acc_ref[...] += bce * vm

    @pl.when(g == _G - 1)
    def _():
        cnt = jnp.sum((cls_ref[...] > 0).astype(jnp.float32))
        denom = cnt * jnp.float32(_HW)
        total = jnp.sum(acc_ref[...])
        out_ref[0, 0] = jnp.where(cnt > 0, total / denom, jnp.float32(0.0))


def kernel(target_masks, target_class_ids, pred_masks):
    pred_v = jnp.transpose(pred_masks, (2, 3, 4, 0, 1)).reshape(_HW * _CB, _R)
    tm_v = jnp.transpose(target_masks, (2, 3, 0, 1)).reshape(_HW * _B, _R)

    loss = pl.pallas_call(
        _loss_kernel,
        grid=(_G,),
        in_specs=[
            pl.BlockSpec((_B, _R), lambda g: (0, 0)),
            pl.BlockSpec((_ROWS // 2, _R), lambda g: (2 * g, 0)),
            pl.BlockSpec((_ROWS // 2, _R), lambda g: (2 * g + 1, 0)),
            pl.BlockSpec((_QR, _R), lambda g: (g, 0)),
        ],
        out_specs=pl.BlockSpec(memory_space=pltpu.SMEM),
        out_shape=jax.ShapeDtypeStruct((1, 1), jnp.float32),
        scratch_shapes=[
            pltpu.VMEM((_CHR, _R), jnp.float32),
            pltpu.VMEM((_CHQ, _R), jnp.float32),
            pltpu.VMEM((_CHQ, _R), jnp.float32),
        ],
    )(target_class_ids, pred_v, pred_v, tm_v)
    return loss[0, 0]
